# baseline (speedup 1.0000x reference)
import jax
import jax.numpy as jnp
from jax.experimental import pallas as pl
from jax.experimental.pallas import tpu as pltpu

_S = 64.0
_MARGIN = 0.5
_C2 = _S * 1.4426950408889634


def _body_factory(B, C, BCC):
    NCB = pl.cdiv(C, BCC)
    NSL = BCC // 8

    def body(x_ref, lab_ref, out_ref, m_ref, s_ref, p_ref):
        j = pl.program_id(0)

        @pl.when(j == 0)
        def _init():
            m_ref[...] = jnp.full((1, B), -jnp.inf, jnp.float32)
            s_ref[...] = jnp.zeros((1, B), jnp.float32)
            p_ref[...] = jnp.zeros((8, B), jnp.float32)

        lab = lab_ref[...]
        rows8 = jax.lax.broadcasted_iota(jnp.int32, (8, B), 0)

        def _slice(k, masked):
            xk = x_ref[pl.ds(k * 8, 8), :]
            if masked:
                gid = j * BCC + k * 8 + rows8
                xk = jnp.where(gid < C, xk, -jnp.inf)
            return xk

        def _block(masked):
            def _p1(k, carry):
                macc, pacc = carry
                xk = _slice(k, masked)
                macc = jnp.maximum(macc, xk)
                pacc = pacc + jnp.where(rows8 == lab - (j * BCC + k * 8),
                                        xk, 0.0)
                return macc, pacc

            macc, pacc = jax.lax.fori_loop(
                0, NSL, _p1,
                (jnp.full((8, B), -jnp.inf, jnp.float32), p_ref[...]),
                unroll=4)
            p_ref[...] = pacc
            m_old = m_ref[...]
            m_new = jnp.maximum(m_old, jnp.max(macc, axis=0, keepdims=True))
            mc = m_new * _C2

            def _p2(k, sacc):
                return sacc + jnp.exp2(_slice(k, masked) * _C2 - mc)

            sacc = jax.lax.fori_loop(0, NSL, _p2,
                                     jnp.zeros((8, B), jnp.float32),
                                     unroll=4)
            bs = jnp.sum(sacc, axis=0, keepdims=True)
            s_ref[...] = s_ref[...] * jnp.exp2(m_old * _C2 - mc) + bs
            m_ref[...] = m_new

        if C % BCC != 0:
            @pl.when(j < NCB - 1)
            def _fast():
                _block(False)

            @pl.when(j == NCB - 1)
            def _slow():
                _block(True)
        else:
            _block(False)

        @pl.when(j == NCB - 1)
        def _finish():
            m = m_ref[...]
            s = s_ref[...]
            a_x = jnp.sum(p_ref[...], axis=0, keepdims=True)
            q = jnp.exp(-_S * _MARGIN)
            ea = jnp.exp2(a_x * _C2 - m * _C2)
            s_adj = jnp.maximum(s - ea * (1.0 - q), ea * q)
            out_ref[...] = _S * m + jnp.log(s_adj) - _S * (a_x - _MARGIN)

    return body, NCB


def _row_losses(cosT, lab2d, BCC):
    C, B = cosT.shape
    body, NCB = _body_factory(B, C, BCC)
    return pl.pallas_call(
        body,
        grid=(NCB,),
        in_specs=[
            pl.BlockSpec((BCC, B), lambda j: (j, 0)),
            pl.BlockSpec((1, B), lambda j: (0, 0)),
        ],
        out_specs=pl.BlockSpec((1, B), lambda j: (0, 0)),
        out_shape=jax.ShapeDtypeStruct((1, B), jnp.float32),
        scratch_shapes=[
            pltpu.VMEM((1, B), jnp.float32),
            pltpu.VMEM((1, B), jnp.float32),
            pltpu.VMEM((8, B), jnp.float32),
        ],
        compiler_params=pltpu.CompilerParams(
            dimension_semantics=("arbitrary",)
        ),
    )(cosT, lab2d)


def _mean_body(x_ref, out_ref):
    B = x_ref.shape[1]
    out_ref[...] = jnp.full((1, 1), jnp.sum(x_ref[...]) * (1.0 / B),
                            jnp.float32)


def _mean_call(row_losses):
    out = pl.pallas_call(
        _mean_body,
        out_shape=jax.ShapeDtypeStruct((1, 1), jnp.float32),
    )(row_losses)
    return out[0, 0]


@jax.jit
def kernel(cosine, label):
    B, _ = cosine.shape
    lab2d = label.astype(jnp.int32).reshape(1, B)
    rl = _row_losses(cosine.T, lab2d, 4096)
    return _mean_call(rl)


# speedup vs baseline: 1.3883x; 1.3883x over previous
import functools

import jax
import jax.numpy as jnp
from jax import lax
from jax.experimental import pallas as pl
from jax.experimental.pallas import tpu as pltpu
from jax.experimental.pallas import tpu_sc as plsc

_S = 64.0
_MARGIN = 0.5
_C2 = _S * 1.4426950408889634

_NUM_SC_CORES = 2
_NUM_SC_SUBCORES = 16
_SC_LANES = 16


def _sc_pick_factory(C, B):
    NW = _NUM_SC_CORES * _NUM_SC_SUBCORES
    bpw = B // NW
    mesh = plsc.VectorSubcoreMesh(core_axis_name="c", subcore_axis_name="s")

    @functools.partial(
        pl.kernel,
        mesh=mesh,
        out_type=jax.ShapeDtypeStruct((B, B), jnp.float32),
        scratch_types=[
            pltpu.VMEM((bpw,), jnp.int32),
            pltpu.VMEM((bpw, B), jnp.float32),
            pltpu.SemaphoreType.DMA,
        ],
    )
    def sc_pick(cosT_hbm, lab_hbm, out_hbm, idx_v, rows_v, sem):
        wid = lax.axis_index("s") * _NUM_SC_CORES + lax.axis_index("c")
        base = wid * bpw
        pltpu.sync_copy(lab_hbm.at[pl.ds(base, bpw)], idx_v)
        pltpu.async_copy(cosT_hbm.at[idx_v], rows_v, sem).wait()
        pltpu.sync_copy(rows_v, out_hbm.at[pl.ds(base, bpw)])

    return sc_pick


def _stream_body_factory(B, C, BCC):
    NCB = pl.cdiv(C, BCC)
    NSL = BCC // 8

    def body(x_ref, m_out_ref, s_out_ref, m_ref, s_ref):
        j = pl.program_id(0)

        @pl.when(j == 0)
        def _init():
            m_ref[...] = jnp.full((1, B), -jnp.inf, jnp.float32)
            s_ref[...] = jnp.zeros((1, B), jnp.float32)

        rows8 = jax.lax.broadcasted_iota(jnp.int32, (8, B), 0)

        def _slice(k, masked):
            xk = x_ref[k * 8:(k + 1) * 8, :]
            if masked:
                gid = j * BCC + k * 8 + rows8
                xk = jnp.where(gid < C, xk, -jnp.inf)
            return xk

        def _block(masked):
            macc = jnp.full((8, B), -jnp.inf, jnp.float32)
            for k in range(NSL):
                macc = jnp.maximum(macc, _slice(k, masked))
            m_old = m_ref[...]
            m_new = jnp.maximum(m_old, jnp.max(macc, axis=0, keepdims=True))
            mc = m_new * _C2
            sacc = jnp.zeros((8, B), jnp.float32)
            for k in range(NSL):
                sacc = sacc + jnp.exp2(_slice(k, masked) * _C2 - mc)
            bs = jnp.sum(sacc, axis=0, keepdims=True)
            s_ref[...] = s_ref[...] * jnp.exp2(m_old * _C2 - mc) + bs
            m_ref[...] = m_new

        if C % BCC != 0:
            @pl.when(j < NCB - 1)
            def _fast():
                _block(False)

            @pl.when(j == NCB - 1)
            def _slow():
                _block(True)
        else:
            _block(False)

        @pl.when(j == NCB - 1)
        def _finish():
            m_out_ref[...] = m_ref[...]
            s_out_ref[...] = s_ref[...]

    return body, NCB


def _stream_call(cosT, BCC):
    C, B = cosT.shape
    body, NCB = _stream_body_factory(B, C, BCC)
    return pl.pallas_call(
        body,
        grid=(NCB,),
        in_specs=[pl.BlockSpec((BCC, B), lambda j: (j, 0))],
        out_specs=[
            pl.BlockSpec((1, B), lambda j: (0, 0)),
            pl.BlockSpec((1, B), lambda j: (0, 0)),
        ],
        out_shape=[
            jax.ShapeDtypeStruct((1, B), jnp.float32),
            jax.ShapeDtypeStruct((1, B), jnp.float32),
        ],
        scratch_shapes=[
            pltpu.VMEM((1, B), jnp.float32),
            pltpu.VMEM((1, B), jnp.float32),
        ],
        compiler_params=pltpu.CompilerParams(
            dimension_semantics=("arbitrary",)
        ),
    )(cosT)


def _combine_body(m_ref, s_ref, rows_ref, out_ref):
    B = m_ref.shape[1]
    m = m_ref[...]
    s = s_ref[...]
    cols = jax.lax.broadcasted_iota(jnp.int32, (8, B), 1)
    rows8 = jax.lax.broadcasted_iota(jnp.int32, (8, B), 0)
    acc = jnp.zeros((8, B), jnp.float32)
    for k in range(B // 8):
        xk = rows_ref[k * 8:(k + 1) * 8, :]
        acc = acc + jnp.where(rows8 + k * 8 == cols, xk, 0.0)
    a_x = jnp.sum(acc, axis=0, keepdims=True)
    q = jnp.exp(-_S * _MARGIN)
    ea = jnp.exp2(a_x * _C2 - m * _C2)
    s_adj = jnp.maximum(s - ea * (1.0 - q), ea * q)
    row_loss = _S * m + jnp.log(s_adj) - _S * (a_x - _MARGIN)
    out_ref[...] = jnp.full((1, 1), jnp.sum(row_loss) * (1.0 / B),
                            jnp.float32)


def _combine_call(m2d, s2d, rows2d):
    out = pl.pallas_call(
        _combine_body,
        out_shape=jax.ShapeDtypeStruct((1, 1), jnp.float32),
    )(m2d, s2d, rows2d)
    return out[0, 0]


@jax.jit
def kernel(cosine, label):
    B, C = cosine.shape
    cosT = cosine.T
    lab1d = label.astype(jnp.int32)
    rows2d = _sc_pick_factory(C, B)(cosT, lab1d)
    m2d, s2d = _stream_call(cosT, 4096)
    return _combine_call(m2d, s2d, rows2d)
